# baseline (speedup 1.0000x reference)
import functools

import jax
import jax.numpy as jnp
from jax import lax
from jax.experimental import pallas as pl
from jax.experimental.pallas import tpu as pltpu
from jax.experimental.pallas import tpu_sc as plsc

N = 10000
E = 320000
D = 128

NC = 2
NS = 16
NW = NC * NS
CH = 128
NCHUNK = 80
NH = 2
HCH = NCHUNK // NH
EPT = CH * NCHUNK
EPAD = EPT * NW
NPAD = 10112
RPT = NPAD // NS
ZFULL = RPT // CH
ZTAIL = RPT - ZFULL * CH
HW = 128
DW = 8
NB = 2
LAG = 1

_mesh = plsc.VectorSubcoreMesh(
    core_axis_name="c", subcore_axis_name="s", num_cores=NC, num_subcores=NS
)


@functools.partial(
    pl.kernel,
    out_type=jax.ShapeDtypeStruct((NC, NPAD, HW), jnp.float32),
    mesh=_mesh,
    scratch_types=[
        pltpu.VMEM((NCHUNK, CH), jnp.int32),
        pltpu.VMEM((CH, HW), jnp.float32),
        pltpu.VMEM_SHARED((NPAD, HW), jnp.float32),
    ],
)
def _deg_kernel(dst_hbm, out_hbm, didx, buf, hist_sh):
    cid = lax.axis_index("c")
    sid = lax.axis_index("s")
    wid = cid * NS + sid
    base = sid * RPT

    @pl.loop(0, CH)
    def _(r):
        buf[r, :] = jnp.zeros((HW,), jnp.float32)

    for k in range(ZFULL):
        pltpu.sync_copy(buf, hist_sh.at[pl.ds(base + k * CH, CH)])
    pltpu.sync_copy(buf.at[pl.ds(0, ZTAIL)],
                    hist_sh.at[pl.ds(base + ZFULL * CH, ZTAIL)])

    @pl.loop(0, CH)
    def _(r):
        buf[r, :] = jnp.ones((HW,), jnp.float32)

    pltpu.sync_copy(dst_hbm.at[wid], didx)
    plsc.subcore_barrier()

    @pl.loop(0, NCHUNK)
    def _(j):
        pltpu.sync_copy(buf, hist_sh.at[didx.at[j]], add=True)

    plsc.subcore_barrier()
    pltpu.sync_copy(
        hist_sh.at[pl.ds(base, RPT)],
        out_hbm.at[cid].at[pl.ds(base, RPT)],
    )


@functools.partial(
    pl.kernel,
    out_type=jax.ShapeDtypeStruct((NC, NPAD, D), jnp.float32),
    mesh=_mesh,
    scratch_types=[
        pltpu.VMEM((HCH, CH), jnp.int32),
        pltpu.VMEM((HCH, CH), jnp.int32),
        pltpu.VMEM((NB, CH, D), jnp.float32),
        pltpu.VMEM_SHARED((NPAD, D), jnp.float32),
        pltpu.SemaphoreType.DMA((NB,)),
        pltpu.SemaphoreType.DMA((NB,)),
    ],
)
def _msg_kernel(h_hbm, src_hbm, dst_hbm, out_hbm, sidx, didx, rows, acc_sh,
                gsem, ssem):
    cid = lax.axis_index("c")
    sid = lax.axis_index("s")
    wid = cid * NS + sid
    base = sid * RPT

    @pl.loop(0, CH)
    def _(r):
        @pl.loop(0, D, step=16)
        def _(c):
            rows[0, r, pl.ds(c, 16)] = jnp.zeros((16,), jnp.float32)

    for k in range(ZFULL):
        pltpu.sync_copy(rows.at[0], acc_sh.at[pl.ds(base + k * CH, CH)])
    pltpu.sync_copy(rows.at[0].at[pl.ds(0, ZTAIL)],
                    acc_sh.at[pl.ds(base + ZFULL * CH, ZTAIL)])

    plsc.subcore_barrier()

    for h in range(NH):
        pltpu.sync_copy(src_hbm.at[wid].at[pl.ds(h * HCH, HCH)], sidx)
        pltpu.sync_copy(dst_hbm.at[wid].at[pl.ds(h * HCH, HCH)], didx)

        for b in range(NB):
            pltpu.async_copy(h_hbm.at[sidx.at[b]], rows.at[b], gsem.at[b])

        @pl.loop(0, HCH)
        def _(i):
            b = lax.rem(i, NB)
            pltpu.make_async_copy(h_hbm.at[sidx.at[i]], rows.at[b],
                                  gsem.at[b]).wait()
            pltpu.async_copy(rows.at[b], acc_sh.at[didx.at[i]], ssem.at[b],
                             add=True)

            k = i - LAG

            @pl.when(jnp.logical_and(k >= 0, k + NB < HCH))
            def _():
                bk = lax.rem(k, NB)
                pltpu.make_async_copy(rows.at[bk], acc_sh.at[didx.at[k]],
                                      ssem.at[bk]).wait()
                pltpu.async_copy(h_hbm.at[sidx.at[k + NB]], rows.at[bk],
                                 gsem.at[bk])

        for k in range(HCH - NB, HCH):
            pltpu.make_async_copy(rows.at[k % NB], acc_sh.at[didx.at[k]],
                                  ssem.at[k % NB]).wait()

    plsc.subcore_barrier()
    pltpu.sync_copy(
        acc_sh.at[pl.ds(base, RPT)],
        out_hbm.at[cid].at[pl.ds(base, RPT)],
    )


def _h1_body(x_ref, w_ref, hist_ref, h_ref, dinv_ref):
    hist = hist_ref[...]
    deg = hist[0, :, 0:1] + hist[1, :, 0:1] + 1.0
    dinv = lax.rsqrt(deg)
    m = jnp.dot(x_ref[...], w_ref[...], preferred_element_type=jnp.float32)
    h_ref[...] = m * dinv
    dinv_ref[...] = jnp.broadcast_to(dinv, (NPAD, DW))


def _mid_body(acc_ref, h1_ref, dinv_ref, b1_ref, w2_ref, h2_ref):
    acc = acc_ref[...]
    d = dinv_ref[...][:, 0:1]
    z = (acc[0] + acc[1] + h1_ref[...]) * d + b1_ref[...]
    a = jnp.where(z > 0, z, jnp.exp(z) - 1.0)
    h2_ref[...] = jnp.dot(a, w2_ref[...], preferred_element_type=jnp.float32) * d


def _out_body(acc_ref, h2_ref, dinv_ref, b2_ref, o_ref):
    acc = acc_ref[...]
    d = dinv_ref[...][:, 0:1]
    z = (acc[0] + acc[1] + h2_ref[...]) * d + b2_ref[...]
    o_ref[...] = jnp.where(z > 0, z, jnp.exp(z) - 1.0)


@jax.jit
def kernel(x, edge_index, W1, b1, W2, b2):
    src = edge_index[0].astype(jnp.int32)
    dst = edge_index[1].astype(jnp.int32)
    pad = jnp.full((EPAD - E,), N, dtype=jnp.int32)
    src3 = jnp.concatenate([src, pad]).reshape(NW, NCHUNK, CH)
    dst3 = jnp.concatenate([dst, pad]).reshape(NW, NCHUNK, CH)
    xp = jnp.concatenate([x, jnp.zeros((NPAD - N, D), x.dtype)], axis=0)
    b1r = b1.reshape(1, D)
    b2r = b2.reshape(1, D)

    hist = _deg_kernel(dst3)
    h1, dinv = pl.pallas_call(
        _h1_body,
        out_shape=(
            jax.ShapeDtypeStruct((NPAD, D), jnp.float32),
            jax.ShapeDtypeStruct((NPAD, DW), jnp.float32),
        ),
    )(xp, W1, hist)
    acc1 = _msg_kernel(h1, src3, dst3)
    h2 = pl.pallas_call(
        _mid_body, out_shape=jax.ShapeDtypeStruct((NPAD, D), jnp.float32)
    )(acc1, h1, dinv, b1r, W2)
    acc2 = _msg_kernel(h2, src3, dst3)
    out = pl.pallas_call(
        _out_body, out_shape=jax.ShapeDtypeStruct((NPAD, D), jnp.float32)
    )(acc2, h2, dinv, b2r)
    return out[:N]


# speedup vs baseline: 1.0937x; 1.0937x over previous
import functools

import jax
import jax.numpy as jnp
from jax import lax
from jax.experimental import pallas as pl
from jax.experimental.pallas import tpu as pltpu
from jax.experimental.pallas import tpu_sc as plsc

N = 10000
E = 320000
D = 128

NC = 2
NS = 16
NW = NC * NS
CH = 128
NCHUNK = 80
TOTCH = NW * NCHUNK
EPAD = TOTCH * CH
XCH0 = 128
XCH1 = 160 - XCH0
NPAD = 10112
RPT = NPAD // NS
ZFULL = RPT // CH
ZTAIL = RPT - ZFULL * CH
HW = 128
DW = 8
NB = 2
LAG = 1

_mesh = plsc.VectorSubcoreMesh(
    core_axis_name="c", subcore_axis_name="s", num_cores=NC, num_subcores=NS
)


@functools.partial(
    pl.kernel,
    out_type=jax.ShapeDtypeStruct((NC, NPAD, HW), jnp.float32),
    mesh=_mesh,
    scratch_types=[
        pltpu.VMEM((NCHUNK, CH), jnp.int32),
        pltpu.VMEM((CH, HW), jnp.float32),
        pltpu.VMEM_SHARED((NPAD, HW), jnp.float32),
    ],
)
def _deg_kernel(dst_hbm, out_hbm, didx, buf, hist_sh):
    cid = lax.axis_index("c")
    sid = lax.axis_index("s")
    wid = cid * NS + sid
    base = sid * RPT

    @pl.loop(0, CH)
    def _(r):
        buf[r, :] = jnp.zeros((HW,), jnp.float32)

    for k in range(ZFULL):
        pltpu.sync_copy(buf, hist_sh.at[pl.ds(base + k * CH, CH)])
    pltpu.sync_copy(buf.at[pl.ds(0, ZTAIL)],
                    hist_sh.at[pl.ds(base + ZFULL * CH, ZTAIL)])

    @pl.loop(0, CH)
    def _(r):
        buf[r, :] = jnp.ones((HW,), jnp.float32)

    pltpu.sync_copy(dst_hbm.at[pl.ds(pl.multiple_of(wid * NCHUNK, 8), NCHUNK)],
                    didx)
    plsc.subcore_barrier()

    @pl.loop(0, NCHUNK)
    def _(j):
        pltpu.sync_copy(buf, hist_sh.at[didx.at[j]], add=True)

    plsc.subcore_barrier()
    pltpu.sync_copy(
        hist_sh.at[pl.ds(base, RPT)],
        out_hbm.at[cid].at[pl.ds(base, RPT)],
    )


@functools.partial(
    pl.kernel,
    out_type=jax.ShapeDtypeStruct((NC, NPAD, D), jnp.float32),
    mesh=_mesh,
    scratch_types=[
        pltpu.VMEM((XCH0 // 2, CH), jnp.int32),
        pltpu.VMEM((XCH0 // 2, CH), jnp.int32),
        pltpu.VMEM((NB, CH, D), jnp.float32),
        pltpu.VMEM_SHARED((NPAD, D), jnp.float32),
        pltpu.SemaphoreType.DMA((NB,)),
        pltpu.SemaphoreType.DMA((NB,)),
    ],
)
def _msg_kernel(h_hbm, src_hbm, dst_hbm, out_hbm, sidx, didx, rows, acc_sh,
                gsem, ssem):
    cid = lax.axis_index("c")
    sid = lax.axis_index("s")
    base = sid * RPT

    @pl.loop(0, CH)
    def _(r):
        @pl.loop(0, D, step=16)
        def _(c):
            rows[0, r, pl.ds(c, 16)] = jnp.zeros((16,), jnp.float32)

    for k in range(ZFULL):
        pltpu.sync_copy(rows.at[0], acc_sh.at[pl.ds(base + k * CH, CH)])
    pltpu.sync_copy(rows.at[0].at[pl.ds(0, ZTAIL)],
                    acc_sh.at[pl.ds(base + ZFULL * CH, ZTAIL)])

    plsc.subcore_barrier()

    def run(start, nch):
        pltpu.sync_copy(src_hbm.at[pl.ds(start, nch)],
                        sidx.at[pl.ds(0, nch)])
        pltpu.sync_copy(dst_hbm.at[pl.ds(start, nch)],
                        didx.at[pl.ds(0, nch)])

        for b in range(NB):
            pltpu.async_copy(h_hbm.at[sidx.at[b]], rows.at[b], gsem.at[b])

        @pl.loop(0, nch)
        def _(i):
            b = lax.rem(i, NB)
            pltpu.make_async_copy(h_hbm.at[sidx.at[i]], rows.at[b],
                                  gsem.at[b]).wait()
            pltpu.async_copy(rows.at[b], acc_sh.at[didx.at[i]], ssem.at[b],
                             add=True)

            k = i - LAG

            @pl.when(jnp.logical_and(k >= 0, k + NB < nch))
            def _():
                bk = lax.rem(k, NB)
                pltpu.make_async_copy(rows.at[bk], acc_sh.at[didx.at[k]],
                                      ssem.at[bk]).wait()
                pltpu.async_copy(h_hbm.at[sidx.at[k + NB]], rows.at[bk],
                                 gsem.at[bk])

        for k in range(nch - NB, nch):
            pltpu.make_async_copy(rows.at[k % NB], acc_sh.at[didx.at[k]],
                                  ssem.at[k % NB]).wait()

    @pl.when(cid == 0)
    def _():
        run(pl.multiple_of(sid * XCH0, 8), XCH0 // 2)
        run(pl.multiple_of(sid * XCH0 + XCH0 // 2, 8), XCH0 // 2)

    @pl.when(cid == 1)
    def _():
        run(pl.multiple_of(NS * XCH0 + sid * XCH1, 8), XCH1)

    plsc.subcore_barrier()
    pltpu.sync_copy(
        acc_sh.at[pl.ds(base, RPT)],
        out_hbm.at[cid].at[pl.ds(base, RPT)],
    )


def _h1_body(x_ref, w_ref, hist_ref, h_ref, dinv_ref):
    hist = hist_ref[...]
    deg = hist[0, :, 0:1] + hist[1, :, 0:1] + 1.0
    dinv = lax.rsqrt(deg)
    m = jnp.dot(x_ref[...], w_ref[...], preferred_element_type=jnp.float32)
    h_ref[...] = m * dinv
    dinv_ref[...] = jnp.broadcast_to(dinv, (NPAD, DW))


def _mid_body(acc_ref, h1_ref, dinv_ref, b1_ref, w2_ref, h2_ref):
    acc = acc_ref[...]
    d = dinv_ref[...][:, 0:1]
    z = (acc[0] + acc[1] + h1_ref[...]) * d + b1_ref[...]
    a = jnp.where(z > 0, z, jnp.exp(z) - 1.0)
    h2_ref[...] = jnp.dot(a, w2_ref[...], preferred_element_type=jnp.float32) * d


def _out_body(acc_ref, h2_ref, dinv_ref, b2_ref, o_ref):
    acc = acc_ref[...]
    d = dinv_ref[...][:, 0:1]
    z = (acc[0] + acc[1] + h2_ref[...]) * d + b2_ref[...]
    o_ref[...] = jnp.where(z > 0, z, jnp.exp(z) - 1.0)


@jax.jit
def kernel(x, edge_index, W1, b1, W2, b2):
    src = edge_index[0].astype(jnp.int32)
    dst = edge_index[1].astype(jnp.int32)
    pad = jnp.full((EPAD - E,), N, dtype=jnp.int32)
    src3 = jnp.concatenate([src, pad]).reshape(TOTCH, CH)
    dst3 = jnp.concatenate([dst, pad]).reshape(TOTCH, CH)
    xp = jnp.concatenate([x, jnp.zeros((NPAD - N, D), x.dtype)], axis=0)
    b1r = b1.reshape(1, D)
    b2r = b2.reshape(1, D)

    hist = _deg_kernel(dst3)
    h1, dinv = pl.pallas_call(
        _h1_body,
        out_shape=(
            jax.ShapeDtypeStruct((NPAD, D), jnp.float32),
            jax.ShapeDtypeStruct((NPAD, DW), jnp.float32),
        ),
    )(xp, W1, hist)
    acc1 = _msg_kernel(h1, src3, dst3)
    h2 = pl.pallas_call(
        _mid_body, out_shape=jax.ShapeDtypeStruct((NPAD, D), jnp.float32)
    )(acc1, h1, dinv, b1r, W2)
    acc2 = _msg_kernel(h2, src3, dst3)
    out = pl.pallas_call(
        _out_body, out_shape=jax.ShapeDtypeStruct((NPAD, D), jnp.float32)
    )(acc2, h2, dinv, b2r)
    return out[:N]


# speedup vs baseline: 1.1642x; 1.0645x over previous
import functools

import jax
import jax.numpy as jnp
from jax import lax
from jax.experimental import pallas as pl
from jax.experimental.pallas import tpu as pltpu
from jax.experimental.pallas import tpu_sc as plsc

N = 10000
E = 320000
D = 128

NC = 2
NS = 16
NW = NC * NS
CH = 128
NCHUNK = 80
TOTCH = NW * NCHUNK
EPAD = TOTCH * CH
HEAVY_CORE = 1
XHEAVY = 128
XLIGHT = 160 - XHEAVY
SEG = XHEAVY // 2
NPAD = 10112
RPT = NPAD // NS
ZFULL = RPT // CH
ZTAIL = RPT - ZFULL * CH
HW = 128
DW = 8
NB = 2
LAG = 1

_mesh = plsc.VectorSubcoreMesh(
    core_axis_name="c", subcore_axis_name="s", num_cores=NC, num_subcores=NS
)


@functools.partial(
    pl.kernel,
    out_type=jax.ShapeDtypeStruct((NC, NPAD, HW), jnp.float32),
    mesh=_mesh,
    scratch_types=[
        pltpu.VMEM((NCHUNK, CH), jnp.int32),
        pltpu.VMEM((CH, HW), jnp.float32),
        pltpu.VMEM_SHARED((NPAD, HW), jnp.float32),
    ],
)
def _deg_kernel(dst_hbm, out_hbm, didx, buf, hist_sh):
    cid = lax.axis_index("c")
    sid = lax.axis_index("s")
    wid = cid * NS + sid
    base = sid * RPT

    @pl.loop(0, CH)
    def _(r):
        buf[r, :] = jnp.zeros((HW,), jnp.float32)

    for k in range(ZFULL):
        pltpu.sync_copy(buf, hist_sh.at[pl.ds(base + k * CH, CH)])
    pltpu.sync_copy(buf.at[pl.ds(0, ZTAIL)],
                    hist_sh.at[pl.ds(base + ZFULL * CH, ZTAIL)])

    @pl.loop(0, CH)
    def _(r):
        buf[r, :] = jnp.ones((HW,), jnp.float32)

    pltpu.sync_copy(dst_hbm.at[pl.ds(pl.multiple_of(wid * NCHUNK, 8), NCHUNK)],
                    didx)
    plsc.subcore_barrier()

    @pl.loop(0, NCHUNK)
    def _(j):
        pltpu.sync_copy(buf, hist_sh.at[didx.at[j]], add=True)

    plsc.subcore_barrier()
    pltpu.sync_copy(
        hist_sh.at[pl.ds(base, RPT)],
        out_hbm.at[cid].at[pl.ds(base, RPT)],
    )


@functools.partial(
    pl.kernel,
    out_type=jax.ShapeDtypeStruct((NC, NPAD, D), jnp.float32),
    mesh=_mesh,
    scratch_types=[
        pltpu.VMEM((SEG, CH), jnp.int32),
        pltpu.VMEM((SEG, CH), jnp.int32),
        pltpu.VMEM((NB, CH, D), jnp.float32),
        pltpu.VMEM_SHARED((NPAD, D), jnp.float32),
        pltpu.SemaphoreType.DMA((NB,)),
        pltpu.SemaphoreType.DMA((NB,)),
    ],
)
def _msg_kernel(h_hbm, src_hbm, dst_hbm, out_hbm, sidx, didx, rows, acc_sh,
                gsem, ssem):
    cid = lax.axis_index("c")
    sid = lax.axis_index("s")
    base = sid * RPT

    @pl.loop(0, CH)
    def _(r):
        @pl.loop(0, D, step=16)
        def _(c):
            rows[0, r, pl.ds(c, 16)] = jnp.zeros((16,), jnp.float32)

    for k in range(ZFULL):
        pltpu.sync_copy(rows.at[0], acc_sh.at[pl.ds(base + k * CH, CH)])
    pltpu.sync_copy(rows.at[0].at[pl.ds(0, ZTAIL)],
                    acc_sh.at[pl.ds(base + ZFULL * CH, ZTAIL)])

    plsc.subcore_barrier()

    def run(start, nch):
        pltpu.sync_copy(src_hbm.at[pl.ds(start, nch)],
                        sidx.at[pl.ds(0, nch)])
        pltpu.sync_copy(dst_hbm.at[pl.ds(start, nch)],
                        didx.at[pl.ds(0, nch)])

        for b in range(NB):
            pltpu.async_copy(h_hbm.at[sidx.at[b]], rows.at[b], gsem.at[b])

        @pl.loop(0, nch)
        def _(i):
            b = lax.rem(i, NB)
            pltpu.make_async_copy(h_hbm.at[sidx.at[i]], rows.at[b],
                                  gsem.at[b]).wait()
            pltpu.async_copy(rows.at[b], acc_sh.at[didx.at[i]], ssem.at[b],
                             add=True)

            k = i - LAG

            @pl.when(jnp.logical_and(k >= 0, k + NB < nch))
            def _():
                bk = lax.rem(k, NB)
                pltpu.make_async_copy(rows.at[bk], acc_sh.at[didx.at[k]],
                                      ssem.at[bk]).wait()
                pltpu.async_copy(h_hbm.at[sidx.at[k + NB]], rows.at[bk],
                                 gsem.at[bk])

        for k in range(nch - NB, nch):
            pltpu.make_async_copy(rows.at[k % NB], acc_sh.at[didx.at[k]],
                                  ssem.at[k % NB]).wait()

    @pl.when(cid == HEAVY_CORE)
    def _():
        run(pl.multiple_of(sid * XHEAVY, 8), SEG)
        run(pl.multiple_of(sid * XHEAVY + SEG, 8), SEG)

    @pl.when(cid != HEAVY_CORE)
    def _():
        run(pl.multiple_of(NS * XHEAVY + sid * XLIGHT, 8), XLIGHT)

    plsc.subcore_barrier()
    pltpu.sync_copy(
        acc_sh.at[pl.ds(base, RPT)],
        out_hbm.at[cid].at[pl.ds(base, RPT)],
    )


def _h1_body(x_ref, w_ref, hist_ref, h_ref, dinv_ref):
    hist = hist_ref[...]
    deg = hist[0, :, 0:1] + hist[1, :, 0:1] + 1.0
    dinv = lax.rsqrt(deg)
    m = jnp.dot(x_ref[...], w_ref[...], preferred_element_type=jnp.float32)
    h_ref[...] = m * dinv
    dinv_ref[...] = jnp.broadcast_to(dinv, (NPAD, DW))


def _mid_body(acc_ref, h1_ref, dinv_ref, b1_ref, w2_ref, h2_ref):
    acc = acc_ref[...]
    d = dinv_ref[...][:, 0:1]
    z = (acc[0] + acc[1] + h1_ref[...]) * d + b1_ref[...]
    a = jnp.where(z > 0, z, jnp.exp(z) - 1.0)
    h2_ref[...] = jnp.dot(a, w2_ref[...], preferred_element_type=jnp.float32) * d


def _out_body(acc_ref, h2_ref, dinv_ref, b2_ref, o_ref):
    acc = acc_ref[...]
    d = dinv_ref[...][:, 0:1]
    z = (acc[0] + acc[1] + h2_ref[...]) * d + b2_ref[...]
    o_ref[...] = jnp.where(z > 0, z, jnp.exp(z) - 1.0)


@jax.jit
def kernel(x, edge_index, W1, b1, W2, b2):
    src = edge_index[0].astype(jnp.int32)
    dst = edge_index[1].astype(jnp.int32)
    pad = jnp.full((EPAD - E,), N, dtype=jnp.int32)
    src3 = jnp.concatenate([src, pad]).reshape(TOTCH, CH)
    dst3 = jnp.concatenate([dst, pad]).reshape(TOTCH, CH)
    xp = jnp.concatenate([x, jnp.zeros((NPAD - N, D), x.dtype)], axis=0)
    b1r = b1.reshape(1, D)
    b2r = b2.reshape(1, D)

    hist = _deg_kernel(dst3)
    h1, dinv = pl.pallas_call(
        _h1_body,
        out_shape=(
            jax.ShapeDtypeStruct((NPAD, D), jnp.float32),
            jax.ShapeDtypeStruct((NPAD, DW), jnp.float32),
        ),
    )(xp, W1, hist)
    acc1 = _msg_kernel(h1, src3, dst3)
    h2 = pl.pallas_call(
        _mid_body, out_shape=jax.ShapeDtypeStruct((NPAD, D), jnp.float32)
    )(acc1, h1, dinv, b1r, W2)
    acc2 = _msg_kernel(h2, src3, dst3)
    out = pl.pallas_call(
        _out_body, out_shape=jax.ShapeDtypeStruct((NPAD, D), jnp.float32)
    )(acc2, h2, dinv, b2r)
    return out[:N]
